# baseline (speedup 1.0000x reference)
import functools
import math

import jax
import jax.numpy as jnp
import numpy as np
from jax import lax
from jax.experimental import pallas as pl
from jax.experimental.pallas import tpu as pltpu
from jax.experimental.pallas import tpu_sc as plsc

_NUM_CORES = 2
_NUM_SUBCORES = 16
_NW = _NUM_CORES * _NUM_SUBCORES

_CHUNK = 320
_BB = 8


def _make_pe(T, d_model):
    position = np.arange(T, dtype=np.float32)[:, None]
    div_term = np.exp(
        np.arange(0, d_model, 2, dtype=np.float32) * (-math.log(10000.0) / d_model)
    )
    pe = np.zeros((T, d_model), dtype=np.float32)
    pe[:, 0::2] = np.sin(position * div_term)
    pe[:, 1::2] = np.cos(position * div_term)
    return pe


def _sc_gather(tables, ids, R):
    rpw = R // _NW
    nchunks = rpw // _CHUNK
    mesh = plsc.VectorSubcoreMesh(core_axis_name="c", subcore_axis_name="s")

    @functools.partial(
        pl.kernel,
        out_type=jax.ShapeDtypeStruct((4, R, 64), jnp.int32),
        mesh=mesh,
        scratch_types=[
            pltpu.VMEM((4 * rpw,), jnp.int32),
            pltpu.VMEM((_CHUNK, 64), jnp.int32),
            pltpu.VMEM((_CHUNK, 64), jnp.int32),
            pltpu.VMEM((_CHUNK, 64), jnp.int32),
            pltpu.VMEM((_CHUNK, 64), jnp.int32),
            pltpu.SemaphoreType.DMA,
            pltpu.SemaphoreType.DMA,
            pltpu.SemaphoreType.DMA,
            pltpu.SemaphoreType.DMA,
            pltpu.SemaphoreType.DMA,
            pltpu.SemaphoreType.DMA,
            pltpu.SemaphoreType.DMA,
            pltpu.SemaphoreType.DMA,
        ],
        compiler_params=pltpu.CompilerParams(use_tc_tiling_on_sc=False),
    )
    def gather_kernel(t0, t1, t2, t3, i0, i1, i2, i3, out, ids_v,
                      r0, r1, r2, r3, g0, g1, g2, g3, s0, s1, s2, s3):
        wid = lax.axis_index("s") * _NUM_CORES + lax.axis_index("c")
        base_w = wid * rpw
        tabs = (t0, t1, t2, t3)
        rows = (r0, r1, r2, r3)
        gsem = (g0, g1, g2, g3)
        ssem = (s0, s1, s2, s3)
        for k, idv in enumerate((i0, i1, i2, i3)):
            pltpu.sync_copy(idv.at[pl.ds(base_w, rpw)],
                            ids_v.at[pl.ds(k * rpw, rpw)])

        def gather_cp(k, c):
            idx = ids_v.at[pl.ds(k * rpw + c * _CHUNK, _CHUNK)]
            return pltpu.make_async_copy(tabs[k].at[idx], rows[k], gsem[k])

        def scatter_cp(k, base):
            return pltpu.make_async_copy(
                rows[k], out.at[k, pl.ds(base, _CHUNK)], ssem[k])

        @pl.loop(0, nchunks)
        def _chunk(c):
            base = base_w + c * _CHUNK
            for k in range(4):
                @pl.when(c > 0)
                def _drain():
                    scatter_cp(k, base).wait()

                gather_cp(k, c).start()
            for k in range(4):
                gather_cp(k, c).wait()
                scatter_cp(k, base).start()

        for k in range(4):
            scatter_cp(k, base_w + (nchunks - 1) * _CHUNK).wait()

    return gather_kernel(*tables, *ids)


def _tc_post(emb, W, b, gamma, beta, pe, B, T):
    d_model = W.shape[1]
    blk = _BB * T
    scale = np.float32(math.sqrt(d_model))
    inv_sqrt2 = np.float32(1.0 / math.sqrt(2.0))

    def body(e0, e1, e2, e3, w, bv, gv, betv, pev, o):
        def unpack(e):
            p = e[0]
            lo = lax.bitcast_convert_type(jnp.left_shift(p, 16), jnp.float32)
            hi = lax.bitcast_convert_type(
                jnp.bitwise_and(p, jnp.int32(-65536)), jnp.float32)
            return jnp.concatenate([lo, hi], axis=-1)

        acc = None
        for k, e in enumerate((e0, e1, e2, e3)):
            part = jnp.dot(unpack(e), w[128 * k:128 * (k + 1)],
                           preferred_element_type=jnp.float32)
            acc = part if acc is None else acc + part
        h = acc + bv[0]
        mu = jnp.mean(h, axis=-1, keepdims=True)
        xc = h - mu
        var = jnp.mean(xc * xc, axis=-1, keepdims=True)
        y = xc * lax.rsqrt(var + 1e-5) * gv[0] + betv[0]
        z = 0.5 * y * (1.0 + lax.erf(y * inv_sqrt2)) * scale
        o[...] = z.reshape(_BB, T, d_model) + pev[None]

    emb_spec = lambda k: pl.BlockSpec((1, blk, 64), lambda i, k=k: (k, i, 0))
    full2d = lambda s: pl.BlockSpec(s, lambda i: (0, 0))
    return pl.pallas_call(
        body,
        grid=(B // _BB,),
        in_specs=[
            emb_spec(0), emb_spec(1), emb_spec(2), emb_spec(3),
            full2d(W.shape), full2d((1, d_model)), full2d((1, d_model)),
            full2d((1, d_model)), full2d((T, d_model)),
        ],
        out_specs=pl.BlockSpec((_BB, T, d_model), lambda i: (i, 0, 0)),
        out_shape=jax.ShapeDtypeStruct((B, T, d_model), jnp.float32),
    )(emb, emb, emb, emb, W, b, gamma, beta, pe)


def kernel(pose_ids, motion_ids, dynamics_ids, face_ids, pose_table,
           motion_table, dynamics_table, face_table, W, b, ln_gamma, ln_beta):
    B, T = pose_ids.shape
    R = B * T
    ids = [x.reshape(-1).astype(jnp.int32)
           for x in (pose_ids, motion_ids, dynamics_ids, face_ids)]
    def _pack(t):
        tt = t.astype(jnp.bfloat16)
        h = t.shape[1] // 2
        return lax.bitcast_convert_type(
            jnp.stack([tt[:, :h], tt[:, h:]], axis=-1), jnp.int32)

    tables = tuple(_pack(t) for t in
                   (pose_table, motion_table, dynamics_table, face_table))
    emb = _sc_gather(tables, ids, R)
    pe = jnp.asarray(_make_pe(T, W.shape[1]))
    return _tc_post(emb, W, b.reshape(1, -1),
                    ln_gamma.reshape(1, -1), ln_beta.reshape(1, -1), pe, B, T)


# speedup vs baseline: 1.1722x; 1.1722x over previous
import functools
import math

import jax
import jax.numpy as jnp
import numpy as np
from jax import lax
from jax.experimental import pallas as pl
from jax.experimental.pallas import tpu as pltpu
from jax.experimental.pallas import tpu_sc as plsc

_NUM_CORES = 2
_NUM_SUBCORES = 16
_NW = _NUM_CORES * _NUM_SUBCORES

_CHUNK = 160
_BB = 8
_SLICES = 4


def _make_pe(T, d_model):
    position = np.arange(T, dtype=np.float32)[:, None]
    div_term = np.exp(
        np.arange(0, d_model, 2, dtype=np.float32) * (-math.log(10000.0) / d_model)
    )
    pe = np.zeros((T, d_model), dtype=np.float32)
    pe[:, 0::2] = np.sin(position * div_term)
    pe[:, 1::2] = np.cos(position * div_term)
    return pe


def _sc_gather(tables, ids, R):
    rpw = R // _NW
    nchunks = rpw // _CHUNK
    mesh = plsc.VectorSubcoreMesh(core_axis_name="c", subcore_axis_name="s")

    @functools.partial(
        pl.kernel,
        out_type=jax.ShapeDtypeStruct((4, R, 128), jnp.float32),
        mesh=mesh,
        scratch_types=[
            pltpu.VMEM((4 * rpw,), jnp.int32),
            pltpu.VMEM((_CHUNK, 128), jnp.float32),
            pltpu.VMEM((_CHUNK, 128), jnp.float32),
            pltpu.VMEM((_CHUNK, 128), jnp.float32),
            pltpu.VMEM((_CHUNK, 128), jnp.float32),
            pltpu.SemaphoreType.DMA,
            pltpu.SemaphoreType.DMA,
            pltpu.SemaphoreType.DMA,
            pltpu.SemaphoreType.DMA,
            pltpu.SemaphoreType.DMA,
            pltpu.SemaphoreType.DMA,
            pltpu.SemaphoreType.DMA,
            pltpu.SemaphoreType.DMA,
        ],
    )
    def gather_kernel(t0, t1, t2, t3, i0, i1, i2, i3, out, ids_v,
                      r0, r1, r2, r3, g0, g1, g2, g3, s0, s1, s2, s3):
        wid = lax.axis_index("s") * _NUM_CORES + lax.axis_index("c")
        base_w = wid * rpw
        tabs = (t0, t1, t2, t3)
        rows = (r0, r1, r2, r3)
        gsem = (g0, g1, g2, g3)
        ssem = (s0, s1, s2, s3)
        for k, idv in enumerate((i0, i1, i2, i3)):
            pltpu.sync_copy(idv.at[pl.ds(base_w, rpw)],
                            ids_v.at[pl.ds(k * rpw, rpw)])

        def gather_cp(k, c):
            idx = ids_v.at[pl.ds(k * rpw + c * _CHUNK, _CHUNK)]
            return pltpu.make_async_copy(tabs[k].at[idx], rows[k], gsem[k])

        def scatter_cp(k, base):
            return pltpu.make_async_copy(
                rows[k], out.at[k, pl.ds(base, _CHUNK)], ssem[k])

        @pl.loop(0, nchunks)
        def _chunk(c):
            base = base_w + c * _CHUNK
            for k in range(4):
                @pl.when(c > 0)
                def _drain():
                    scatter_cp(k, base).wait()

                gather_cp(k, c).start()
            for k in range(4):
                gather_cp(k, c).wait()
                scatter_cp(k, base).start()

        for k in range(4):
            scatter_cp(k, base_w + (nchunks - 1) * _CHUNK).wait()

    return gather_kernel(*tables, *ids)


def _tc_post_slice(carry, emb, W, b, gamma, beta, pe, B, T, s, n_s):
    d_model = W.shape[1]
    blk = _BB * T
    bs = B // n_s
    steps = bs // _BB
    scale = np.float32(math.sqrt(d_model))
    inv_sqrt2 = np.float32(1.0 / math.sqrt(2.0))

    def body(_, e0, e1, e2, e3, w, bv, gv, betv, pev, o):
        acc = jnp.dot(e0[0], w[0:128], preferred_element_type=jnp.float32)
        acc = acc + jnp.dot(e1[0], w[128:256], preferred_element_type=jnp.float32)
        acc = acc + jnp.dot(e2[0], w[256:384], preferred_element_type=jnp.float32)
        acc = acc + jnp.dot(e3[0], w[384:512], preferred_element_type=jnp.float32)
        h = acc + bv[0]
        mu = jnp.mean(h, axis=-1, keepdims=True)
        xc = h - mu
        var = jnp.mean(xc * xc, axis=-1, keepdims=True)
        y = xc * lax.rsqrt(var + 1e-5) * gv[0] + betv[0]
        z = 0.5 * y * (1.0 + lax.erf(y * inv_sqrt2)) * scale
        o[...] = z.reshape(_BB, T, d_model) + pev[None]

    emb_spec = lambda k: pl.BlockSpec((1, blk, 128), lambda i, k=k: (k, i, 0))
    full2d = lambda sh: pl.BlockSpec(sh, lambda i: (0, 0))
    return pl.pallas_call(
        body,
        grid=(steps,),
        in_specs=[
            pl.BlockSpec(memory_space=pl.ANY),
            emb_spec(0), emb_spec(1), emb_spec(2), emb_spec(3),
            full2d(W.shape), full2d((1, d_model)), full2d((1, d_model)),
            full2d((1, d_model)), full2d((T, d_model)),
        ],
        out_specs=pl.BlockSpec((_BB, T, d_model),
                               lambda i, s=s, steps=steps: (s * steps + i, 0, 0)),
        out_shape=jax.ShapeDtypeStruct((B, T, d_model), jnp.float32),
        input_output_aliases={0: 0},
    )(carry, emb, emb, emb, emb, W, b, gamma, beta, pe)


def kernel(pose_ids, motion_ids, dynamics_ids, face_ids, pose_table,
           motion_table, dynamics_table, face_table, W, b, ln_gamma, ln_beta):
    B, T = pose_ids.shape
    R = B * T
    rs = R // _SLICES
    ids = [x.reshape(-1).astype(jnp.int32)
           for x in (pose_ids, motion_ids, dynamics_ids, face_ids)]
    tables = (pose_table, motion_table, dynamics_table, face_table)
    pe = jnp.asarray(_make_pe(T, W.shape[1]))
    b2, g2, be2 = b.reshape(1, -1), ln_gamma.reshape(1, -1), ln_beta.reshape(1, -1)

    out = jnp.zeros((B, T, W.shape[1]), jnp.float32)
    for s in range(_SLICES):
        ids_s = [i[s * rs:(s + 1) * rs] for i in ids]
        emb_s = _sc_gather(tables, ids_s, rs)
        out = _tc_post_slice(out, emb_s, W, b2, g2, be2, pe, B, T, s, _SLICES)
    return out


# speedup vs baseline: 1.4375x; 1.2263x over previous
import functools
import math

import jax
import jax.numpy as jnp
import numpy as np
from jax import lax
from jax.experimental import pallas as pl
from jax.experimental.pallas import tpu as pltpu
from jax.experimental.pallas import tpu_sc as plsc

_NUM_CORES = 2
_NUM_SUBCORES = 16
_NW = _NUM_CORES * _NUM_SUBCORES

_CHUNK = 160
_BB = 8
_SLICES = 4


def _make_pe(T, d_model):
    position = np.arange(T, dtype=np.float32)[:, None]
    div_term = np.exp(
        np.arange(0, d_model, 2, dtype=np.float32) * (-math.log(10000.0) / d_model)
    )
    pe = np.zeros((T, d_model), dtype=np.float32)
    pe[:, 0::2] = np.sin(position * div_term)
    pe[:, 1::2] = np.cos(position * div_term)
    return pe


def _sc_gather(tables, ids, R):
    rpw = R // _NW
    nchunks = rpw // _CHUNK
    mesh = plsc.VectorSubcoreMesh(core_axis_name="c", subcore_axis_name="s")

    @functools.partial(
        pl.kernel,
        out_type=jax.ShapeDtypeStruct((4, R, 128), jnp.float32),
        mesh=mesh,
        scratch_types=[
            pltpu.VMEM((4 * rpw,), jnp.int32),
            pltpu.VMEM((_CHUNK, 128), jnp.float32),
            pltpu.VMEM((_CHUNK, 128), jnp.float32),
            pltpu.VMEM((_CHUNK, 128), jnp.float32),
            pltpu.VMEM((_CHUNK, 128), jnp.float32),
            pltpu.SemaphoreType.DMA,
            pltpu.SemaphoreType.DMA,
            pltpu.SemaphoreType.DMA,
            pltpu.SemaphoreType.DMA,
            pltpu.SemaphoreType.DMA,
            pltpu.SemaphoreType.DMA,
            pltpu.SemaphoreType.DMA,
            pltpu.SemaphoreType.DMA,
        ],
    )
    def gather_kernel(t0, t1, t2, t3, i0, i1, i2, i3, out, ids_v,
                      r0, r1, r2, r3, g0, g1, g2, g3, s0, s1, s2, s3):
        wid = lax.axis_index("s") * _NUM_CORES + lax.axis_index("c")
        base_w = wid * rpw
        tabs = (t0, t1, t2, t3)
        rows = (r0, r1, r2, r3)
        gsem = (g0, g1, g2, g3)
        ssem = (s0, s1, s2, s3)
        for k, idv in enumerate((i0, i1, i2, i3)):
            pltpu.sync_copy(idv.at[pl.ds(base_w, rpw)],
                            ids_v.at[pl.ds(k * rpw, rpw)])

        def gather_cp(k, c):
            idx = ids_v.at[pl.ds(k * rpw + c * _CHUNK, _CHUNK)]
            return pltpu.make_async_copy(tabs[k].at[idx], rows[k], gsem[k])

        def scatter_cp(k, base):
            return pltpu.make_async_copy(
                rows[k], out.at[k, pl.ds(base, _CHUNK)], ssem[k])

        @pl.loop(0, nchunks)
        def _chunk(c):
            base = base_w + c * _CHUNK
            for k in range(4):
                @pl.when(c > 0)
                def _drain():
                    scatter_cp(k, base).wait()

                gather_cp(k, c).start()
            for k in range(4):
                gather_cp(k, c).wait()
                scatter_cp(k, base).start()

        for k in range(4):
            scatter_cp(k, base_w + (nchunks - 1) * _CHUNK).wait()

    return gather_kernel(*tables, *ids)


def _tc_post_slice(carry, emb, W, b, gamma, beta, pe, B, T, s, n_s):
    d_model = W.shape[1]
    blk = _BB * T
    bs = B // n_s
    steps = bs // _BB
    scale = np.float32(math.sqrt(d_model))
    inv_sqrt2 = np.float32(1.0 / math.sqrt(2.0))

    def body(_, e0, e1, e2, e3, w, bv, gv, betv, pev, o):
        acc = jnp.dot(e0[0], w[0:128], preferred_element_type=jnp.float32)
        acc = acc + jnp.dot(e1[0], w[128:256], preferred_element_type=jnp.float32)
        acc = acc + jnp.dot(e2[0], w[256:384], preferred_element_type=jnp.float32)
        acc = acc + jnp.dot(e3[0], w[384:512], preferred_element_type=jnp.float32)
        h = acc + bv[0]
        mu = jnp.mean(h, axis=-1, keepdims=True)
        xc = h - mu
        var = jnp.mean(xc * xc, axis=-1, keepdims=True)
        y = xc * lax.rsqrt(var + 1e-5) * gv[0] + betv[0]
        z = 0.5 * y * (1.0 + lax.erf(y * inv_sqrt2)) * scale
        o[...] = z.reshape(_BB, T, d_model) + pev[None]

    emb_spec = lambda k: pl.BlockSpec((1, blk, 128), lambda i, k=k: (k, i, 0))
    full2d = lambda sh: pl.BlockSpec(sh, lambda i: (0, 0))
    data_specs = [
        emb_spec(0), emb_spec(1), emb_spec(2), emb_spec(3),
        full2d(W.shape), full2d((1, d_model)), full2d((1, d_model)),
        full2d((1, d_model)), full2d((T, d_model)),
    ]
    data_args = (emb, emb, emb, emb, W, b, gamma, beta, pe)
    if carry is None:
        in_specs, args, aliases = data_specs, data_args, {}
        tc_body = lambda *refs: body(None, *refs)
    else:
        in_specs = [pl.BlockSpec(memory_space=pl.ANY)] + data_specs
        args = (carry,) + data_args
        aliases = {0: 0}
        tc_body = body
    return pl.pallas_call(
        tc_body,
        grid=(steps,),
        in_specs=in_specs,
        out_specs=pl.BlockSpec((_BB, T, d_model),
                               lambda i, s=s, steps=steps: (s * steps + i, 0, 0)),
        out_shape=jax.ShapeDtypeStruct((B, T, d_model), jnp.float32),
        input_output_aliases=aliases,
    )(*args)


def kernel(pose_ids, motion_ids, dynamics_ids, face_ids, pose_table,
           motion_table, dynamics_table, face_table, W, b, ln_gamma, ln_beta):
    B, T = pose_ids.shape
    R = B * T
    rs = R // _SLICES
    ids = [x.reshape(-1).astype(jnp.int32)
           for x in (pose_ids, motion_ids, dynamics_ids, face_ids)]
    tables = (pose_table, motion_table, dynamics_table, face_table)
    pe = jnp.asarray(_make_pe(T, W.shape[1]))
    b2, g2, be2 = b.reshape(1, -1), ln_gamma.reshape(1, -1), ln_beta.reshape(1, -1)

    embs = []
    for s in range(_SLICES):
        ids_s = [i[s * rs:(s + 1) * rs] for i in ids]
        embs.append(_sc_gather(tables, ids_s, rs))
    out = None
    for s in range(_SLICES):
        out = _tc_post_slice(out, embs[s], W, b2, g2, be2, pe, B, T, s, _SLICES)
    return out


# speedup vs baseline: 1.9582x; 1.3623x over previous
import functools
import math

import jax
import jax.numpy as jnp
import numpy as np
from jax import lax
from jax.experimental import pallas as pl
from jax.experimental.pallas import tpu as pltpu
from jax.experimental.pallas import tpu_sc as plsc

_NUM_CORES = 2
_NUM_SUBCORES = 16
_NW = _NUM_CORES * _NUM_SUBCORES

_CHUNK = 160
_BB = 8
_SLICES = 4


def _make_pe(T, d_model):
    position = np.arange(T, dtype=np.float32)[:, None]
    div_term = np.exp(
        np.arange(0, d_model, 2, dtype=np.float32) * (-math.log(10000.0) / d_model)
    )
    pe = np.zeros((T, d_model), dtype=np.float32)
    pe[:, 0::2] = np.sin(position * div_term)
    pe[:, 1::2] = np.cos(position * div_term)
    return pe


def _sc_gather(tables, ids, R):
    rpw = R // _NW
    nchunks = rpw // _CHUNK
    mesh = plsc.VectorSubcoreMesh(core_axis_name="c", subcore_axis_name="s")

    @functools.partial(
        pl.kernel,
        out_type=jax.ShapeDtypeStruct((4, R, 128), jnp.float32),
        mesh=mesh,
        scratch_types=[
            pltpu.VMEM_SHARED((1024, 128), jnp.float32),
            pltpu.VMEM_SHARED((512, 128), jnp.float32),
            pltpu.VMEM_SHARED((256, 128), jnp.float32),
            pltpu.VMEM_SHARED((256, 128), jnp.float32),
            pltpu.VMEM((4 * rpw,), jnp.int32),
            pltpu.VMEM((_CHUNK, 128), jnp.float32),
            pltpu.VMEM((_CHUNK, 128), jnp.float32),
            pltpu.VMEM((_CHUNK, 128), jnp.float32),
            pltpu.VMEM((_CHUNK, 128), jnp.float32),
            pltpu.SemaphoreType.DMA,
            pltpu.SemaphoreType.DMA,
            pltpu.SemaphoreType.DMA,
            pltpu.SemaphoreType.DMA,
            pltpu.SemaphoreType.DMA,
            pltpu.SemaphoreType.DMA,
            pltpu.SemaphoreType.DMA,
            pltpu.SemaphoreType.DMA,
        ],
    )
    def gather_kernel(t0, t1, t2, t3, i0, i1, i2, i3, out,
                      sh0, sh1, sh2, sh3, ids_v,
                      r0, r1, r2, r3, g0, g1, g2, g3, s0, s1, s2, s3):
        wid = lax.axis_index("s") * _NUM_CORES + lax.axis_index("c")
        base_w = wid * rpw
        shs = (sh0, sh1, sh2, sh3)
        rows = (r0, r1, r2, r3)
        gsem = (g0, g1, g2, g3)
        ssem = (s0, s1, s2, s3)
        @pl.when(lax.axis_index("s") == 0)
        def _stage():
            for tab, sh in zip((t0, t1, t2, t3), shs):
                pltpu.sync_copy(tab, sh)

        for k, idv in enumerate((i0, i1, i2, i3)):
            pltpu.sync_copy(idv.at[pl.ds(base_w, rpw)],
                            ids_v.at[pl.ds(k * rpw, rpw)])
        plsc.subcore_barrier()

        def gather_cp(k, c):
            idx = ids_v.at[pl.ds(k * rpw + c * _CHUNK, _CHUNK)]
            return pltpu.make_async_copy(shs[k].at[idx], rows[k], gsem[k])

        def scatter_cp(k, base):
            return pltpu.make_async_copy(
                rows[k], out.at[k, pl.ds(base, _CHUNK)], ssem[k])

        @pl.loop(0, nchunks)
        def _chunk(c):
            base = base_w + c * _CHUNK
            for k in range(4):
                @pl.when(c > 0)
                def _drain():
                    scatter_cp(k, base).wait()

                gather_cp(k, c).start()
            for k in range(4):
                gather_cp(k, c).wait()
                scatter_cp(k, base).start()

        for k in range(4):
            scatter_cp(k, base_w + (nchunks - 1) * _CHUNK).wait()

    return gather_kernel(*tables, *ids)


def _tc_post_slice(carry, emb, W, b, gamma, beta, pe, B, T, s, n_s):
    d_model = W.shape[1]
    blk = _BB * T
    bs = B // n_s
    steps = bs // _BB
    scale = np.float32(math.sqrt(d_model))
    inv_sqrt2 = np.float32(1.0 / math.sqrt(2.0))

    def body(_, e0, e1, e2, e3, w, bv, gv, betv, pev, o):
        acc = jnp.dot(e0[0], w[0:128], preferred_element_type=jnp.float32)
        acc = acc + jnp.dot(e1[0], w[128:256], preferred_element_type=jnp.float32)
        acc = acc + jnp.dot(e2[0], w[256:384], preferred_element_type=jnp.float32)
        acc = acc + jnp.dot(e3[0], w[384:512], preferred_element_type=jnp.float32)
        h = acc + bv[0]
        mu = jnp.mean(h, axis=-1, keepdims=True)
        xc = h - mu
        var = jnp.mean(xc * xc, axis=-1, keepdims=True)
        y = xc * lax.rsqrt(var + 1e-5) * gv[0] + betv[0]
        z = 0.5 * y * (1.0 + lax.erf(y * inv_sqrt2)) * scale
        o[...] = z.reshape(_BB, T, d_model) + pev[None]

    emb_spec = lambda k: pl.BlockSpec((1, blk, 128), lambda i, k=k: (k, i, 0))
    full2d = lambda sh: pl.BlockSpec(sh, lambda i: (0, 0))
    data_specs = [
        emb_spec(0), emb_spec(1), emb_spec(2), emb_spec(3),
        full2d(W.shape), full2d((1, d_model)), full2d((1, d_model)),
        full2d((1, d_model)), full2d((T, d_model)),
    ]
    data_args = (emb, emb, emb, emb, W, b, gamma, beta, pe)
    if carry is None:
        in_specs, args, aliases = data_specs, data_args, {}
        tc_body = lambda *refs: body(None, *refs)
    else:
        in_specs = [pl.BlockSpec(memory_space=pl.ANY)] + data_specs
        args = (carry,) + data_args
        aliases = {0: 0}
        tc_body = body
    return pl.pallas_call(
        tc_body,
        grid=(steps,),
        in_specs=in_specs,
        out_specs=pl.BlockSpec((_BB, T, d_model),
                               lambda i, s=s, steps=steps: (s * steps + i, 0, 0)),
        out_shape=jax.ShapeDtypeStruct((B, T, d_model), jnp.float32),
        input_output_aliases=aliases,
    )(*args)


def kernel(pose_ids, motion_ids, dynamics_ids, face_ids, pose_table,
           motion_table, dynamics_table, face_table, W, b, ln_gamma, ln_beta):
    B, T = pose_ids.shape
    R = B * T
    rs = R // _SLICES
    ids = [x.reshape(-1).astype(jnp.int32)
           for x in (pose_ids, motion_ids, dynamics_ids, face_ids)]
    tables = (pose_table, motion_table, dynamics_table, face_table)
    pe = jnp.asarray(_make_pe(T, W.shape[1]))
    b2, g2, be2 = b.reshape(1, -1), ln_gamma.reshape(1, -1), ln_beta.reshape(1, -1)

    embs = []
    for s in range(_SLICES):
        ids_s = [i[s * rs:(s + 1) * rs] for i in ids]
        embs.append(_sc_gather(tables, ids_s, rs))
    out = None
    for s in range(_SLICES):
        out = _tc_post_slice(out, embs[s], W, b2, g2, be2, pe, B, T, s, _SLICES)
    return out


# speedup vs baseline: 1.9957x; 1.0191x over previous
import functools
import math

import jax
import jax.numpy as jnp
import numpy as np
from jax import lax
from jax.experimental import pallas as pl
from jax.experimental.pallas import tpu as pltpu
from jax.experimental.pallas import tpu_sc as plsc

_NUM_CORES = 2
_NUM_SUBCORES = 16
_NW = _NUM_CORES * _NUM_SUBCORES

_CHUNK = 160
_BB = 16
_SLICES = 8


def _make_pe(T, d_model):
    position = np.arange(T, dtype=np.float32)[:, None]
    div_term = np.exp(
        np.arange(0, d_model, 2, dtype=np.float32) * (-math.log(10000.0) / d_model)
    )
    pe = np.zeros((T, d_model), dtype=np.float32)
    pe[:, 0::2] = np.sin(position * div_term)
    pe[:, 1::2] = np.cos(position * div_term)
    return pe


def _sc_gather(tables, ids, R):
    rpw = R // _NW
    nchunks = rpw // _CHUNK
    mesh = plsc.VectorSubcoreMesh(core_axis_name="c", subcore_axis_name="s")

    @functools.partial(
        pl.kernel,
        out_type=jax.ShapeDtypeStruct((4, R, 128), jnp.float32),
        mesh=mesh,
        scratch_types=[
            pltpu.VMEM_SHARED((1024, 128), jnp.float32),
            pltpu.VMEM_SHARED((512, 128), jnp.float32),
            pltpu.VMEM_SHARED((256, 128), jnp.float32),
            pltpu.VMEM_SHARED((256, 128), jnp.float32),
            pltpu.VMEM((4 * rpw,), jnp.int32),
            pltpu.VMEM((_CHUNK, 128), jnp.float32),
            pltpu.VMEM((_CHUNK, 128), jnp.float32),
            pltpu.VMEM((_CHUNK, 128), jnp.float32),
            pltpu.VMEM((_CHUNK, 128), jnp.float32),
            pltpu.SemaphoreType.DMA,
            pltpu.SemaphoreType.DMA,
            pltpu.SemaphoreType.DMA,
            pltpu.SemaphoreType.DMA,
            pltpu.SemaphoreType.DMA,
            pltpu.SemaphoreType.DMA,
            pltpu.SemaphoreType.DMA,
            pltpu.SemaphoreType.DMA,
        ],
    )
    def gather_kernel(t0, t1, t2, t3, i0, i1, i2, i3, out,
                      sh0, sh1, sh2, sh3, ids_v,
                      r0, r1, r2, r3, g0, g1, g2, g3, s0, s1, s2, s3):
        wid = lax.axis_index("s") * _NUM_CORES + lax.axis_index("c")
        base_w = wid * rpw
        shs = (sh0, sh1, sh2, sh3)
        rows = (r0, r1, r2, r3)
        gsem = (g0, g1, g2, g3)
        ssem = (s0, s1, s2, s3)
        @pl.when(lax.axis_index("s") == 0)
        def _stage():
            for tab, sh in zip((t0, t1, t2, t3), shs):
                pltpu.sync_copy(tab, sh)

        for k, idv in enumerate((i0, i1, i2, i3)):
            pltpu.sync_copy(idv.at[pl.ds(base_w, rpw)],
                            ids_v.at[pl.ds(k * rpw, rpw)])
        plsc.subcore_barrier()

        def gather_cp(k, c):
            idx = ids_v.at[pl.ds(k * rpw + c * _CHUNK, _CHUNK)]
            return pltpu.make_async_copy(shs[k].at[idx], rows[k], gsem[k])

        def scatter_cp(k, base):
            return pltpu.make_async_copy(
                rows[k], out.at[k, pl.ds(base, _CHUNK)], ssem[k])

        @pl.loop(0, nchunks)
        def _chunk(c):
            base = base_w + c * _CHUNK
            for k in range(4):
                @pl.when(c > 0)
                def _drain():
                    scatter_cp(k, base).wait()

                gather_cp(k, c).start()
            for k in range(4):
                gather_cp(k, c).wait()
                scatter_cp(k, base).start()

        for k in range(4):
            scatter_cp(k, base_w + (nchunks - 1) * _CHUNK).wait()

    return gather_kernel(*tables, *ids)


def _tc_post_slice(carry, emb, W, b, gamma, beta, pe, B, T, s, n_s):
    d_model = W.shape[1]
    blk = _BB * T
    bs = B // n_s
    steps = bs // _BB
    scale = np.float32(math.sqrt(d_model))
    inv_sqrt2 = np.float32(1.0 / math.sqrt(2.0))

    def body(_, e0, e1, e2, e3, w, bv, gv, betv, pev, o):
        acc = jnp.dot(e0[0], w[0:128], preferred_element_type=jnp.float32)
        acc = acc + jnp.dot(e1[0], w[128:256], preferred_element_type=jnp.float32)
        acc = acc + jnp.dot(e2[0], w[256:384], preferred_element_type=jnp.float32)
        acc = acc + jnp.dot(e3[0], w[384:512], preferred_element_type=jnp.float32)
        h = acc + bv[0]
        mu = jnp.mean(h, axis=-1, keepdims=True)
        xc = h - mu
        var = jnp.mean(xc * xc, axis=-1, keepdims=True)
        y = xc * lax.rsqrt(var + 1e-5) * gv[0] + betv[0]
        z = 0.5 * y * (1.0 + lax.erf(y * inv_sqrt2)) * scale
        o[...] = z.reshape(_BB, T, d_model) + pev[None]

    emb_spec = lambda k: pl.BlockSpec((1, blk, 128), lambda i, k=k: (k, i, 0))
    full2d = lambda sh: pl.BlockSpec(sh, lambda i: (0, 0))
    data_specs = [
        emb_spec(0), emb_spec(1), emb_spec(2), emb_spec(3),
        full2d(W.shape), full2d((1, d_model)), full2d((1, d_model)),
        full2d((1, d_model)), full2d((T, d_model)),
    ]
    data_args = (emb, emb, emb, emb, W, b, gamma, beta, pe)
    if carry is None:
        in_specs, args, aliases = data_specs, data_args, {}
        tc_body = lambda *refs: body(None, *refs)
    else:
        in_specs = [pl.BlockSpec(memory_space=pl.ANY)] + data_specs
        args = (carry,) + data_args
        aliases = {0: 0}
        tc_body = body
    return pl.pallas_call(
        tc_body,
        grid=(steps,),
        in_specs=in_specs,
        out_specs=pl.BlockSpec((_BB, T, d_model),
                               lambda i, s=s, steps=steps: (s * steps + i, 0, 0)),
        out_shape=jax.ShapeDtypeStruct((B, T, d_model), jnp.float32),
        input_output_aliases=aliases,
    )(*args)


def kernel(pose_ids, motion_ids, dynamics_ids, face_ids, pose_table,
           motion_table, dynamics_table, face_table, W, b, ln_gamma, ln_beta):
    B, T = pose_ids.shape
    R = B * T
    rs = R // _SLICES
    ids = [x.reshape(-1).astype(jnp.int32)
           for x in (pose_ids, motion_ids, dynamics_ids, face_ids)]
    tables = (pose_table, motion_table, dynamics_table, face_table)
    pe = jnp.asarray(_make_pe(T, W.shape[1]))
    b2, g2, be2 = b.reshape(1, -1), ln_gamma.reshape(1, -1), ln_beta.reshape(1, -1)

    embs = []
    for s in range(_SLICES):
        ids_s = [i[s * rs:(s + 1) * rs] for i in ids]
        embs.append(_sc_gather(tables, ids_s, rs))
    out = None
    for s in range(_SLICES):
        out = _tc_post_slice(out, embs[s], W, b2, g2, be2, pe, B, T, s, _SLICES)
    return out


# speedup vs baseline: 2.0349x; 1.0196x over previous
import functools
import math

import jax
import jax.numpy as jnp
import numpy as np
from jax import lax
from jax.experimental import pallas as pl
from jax.experimental.pallas import tpu as pltpu
from jax.experimental.pallas import tpu_sc as plsc

_NUM_CORES = 2
_NUM_SUBCORES = 16
_NW = _NUM_CORES * _NUM_SUBCORES

_CHUNK = 160
_BB = 16
_SLICES = 4


def _make_pe(T, d_model):
    position = np.arange(T, dtype=np.float32)[:, None]
    div_term = np.exp(
        np.arange(0, d_model, 2, dtype=np.float32) * (-math.log(10000.0) / d_model)
    )
    pe = np.zeros((T, d_model), dtype=np.float32)
    pe[:, 0::2] = np.sin(position * div_term)
    pe[:, 1::2] = np.cos(position * div_term)
    return pe


def _sc_gather(tables, ids, R):
    rpw = R // _NW
    nchunks = rpw // _CHUNK
    mesh = plsc.VectorSubcoreMesh(core_axis_name="c", subcore_axis_name="s")

    @functools.partial(
        pl.kernel,
        out_type=jax.ShapeDtypeStruct((4, R, 128), jnp.float32),
        mesh=mesh,
        scratch_types=[
            pltpu.VMEM_SHARED((1024, 128), jnp.float32),
            pltpu.VMEM_SHARED((512, 128), jnp.float32),
            pltpu.VMEM_SHARED((256, 128), jnp.float32),
            pltpu.VMEM_SHARED((256, 128), jnp.float32),
            pltpu.VMEM((4 * rpw,), jnp.int32),
            pltpu.VMEM((_CHUNK, 128), jnp.float32),
            pltpu.VMEM((_CHUNK, 128), jnp.float32),
            pltpu.VMEM((_CHUNK, 128), jnp.float32),
            pltpu.VMEM((_CHUNK, 128), jnp.float32),
            pltpu.SemaphoreType.DMA,
            pltpu.SemaphoreType.DMA,
            pltpu.SemaphoreType.DMA,
            pltpu.SemaphoreType.DMA,
            pltpu.SemaphoreType.DMA,
            pltpu.SemaphoreType.DMA,
            pltpu.SemaphoreType.DMA,
            pltpu.SemaphoreType.DMA,
        ],
    )
    def gather_kernel(t0, t1, t2, t3, i0, i1, i2, i3, out,
                      sh0, sh1, sh2, sh3, ids_v,
                      r0, r1, r2, r3, g0, g1, g2, g3, s0, s1, s2, s3):
        wid = lax.axis_index("s") * _NUM_CORES + lax.axis_index("c")
        base_w = wid * rpw
        shs = (sh0, sh1, sh2, sh3)
        rows = (r0, r1, r2, r3)
        gsem = (g0, g1, g2, g3)
        ssem = (s0, s1, s2, s3)
        @pl.when(lax.axis_index("s") == 0)
        def _stage():
            for tab, sh in zip((t0, t1, t2, t3), shs):
                pltpu.sync_copy(tab, sh)

        for k, idv in enumerate((i0, i1, i2, i3)):
            pltpu.sync_copy(idv.at[pl.ds(base_w, rpw)],
                            ids_v.at[pl.ds(k * rpw, rpw)])
        plsc.subcore_barrier()

        def gather_cp(k, c):
            idx = ids_v.at[pl.ds(k * rpw + c * _CHUNK, _CHUNK)]
            return pltpu.make_async_copy(shs[k].at[idx], rows[k], gsem[k])

        def scatter_cp(k, base):
            return pltpu.make_async_copy(
                rows[k], out.at[k, pl.ds(base, _CHUNK)], ssem[k])

        @pl.loop(0, nchunks)
        def _chunk(c):
            base = base_w + c * _CHUNK
            for k in range(4):
                @pl.when(c > 0)
                def _drain():
                    scatter_cp(k, base).wait()

                gather_cp(k, c).start()
            for k in range(4):
                gather_cp(k, c).wait()
                scatter_cp(k, base).start()

        for k in range(4):
            scatter_cp(k, base_w + (nchunks - 1) * _CHUNK).wait()

    return gather_kernel(*tables, *ids)


def _tc_post_slice(carry, emb, W, b, gamma, beta, pe, B, T, s, n_s):
    d_model = W.shape[1]
    blk = _BB * T
    bs = B // n_s
    steps = bs // _BB
    scale = np.float32(math.sqrt(d_model))
    inv_sqrt2 = np.float32(1.0 / math.sqrt(2.0))

    def body(_, e0, e1, e2, e3, w, bv, gv, betv, pev, o):
        acc = jnp.dot(e0[0], w[0:128], preferred_element_type=jnp.float32)
        acc = acc + jnp.dot(e1[0], w[128:256], preferred_element_type=jnp.float32)
        acc = acc + jnp.dot(e2[0], w[256:384], preferred_element_type=jnp.float32)
        acc = acc + jnp.dot(e3[0], w[384:512], preferred_element_type=jnp.float32)
        h = acc + bv[0]
        mu = jnp.mean(h, axis=-1, keepdims=True)
        xc = h - mu
        var = jnp.mean(xc * xc, axis=-1, keepdims=True)
        y = xc * lax.rsqrt(var + 1e-5) * gv[0] + betv[0]
        z = 0.5 * y * (1.0 + lax.erf(y * inv_sqrt2)) * scale
        o[...] = z.reshape(_BB, T, d_model) + pev[None]

    emb_spec = lambda k: pl.BlockSpec((1, blk, 128), lambda i, k=k: (k, i, 0))
    full2d = lambda sh: pl.BlockSpec(sh, lambda i: (0, 0))
    data_specs = [
        emb_spec(0), emb_spec(1), emb_spec(2), emb_spec(3),
        full2d(W.shape), full2d((1, d_model)), full2d((1, d_model)),
        full2d((1, d_model)), full2d((T, d_model)),
    ]
    data_args = (emb, emb, emb, emb, W, b, gamma, beta, pe)
    if carry is None:
        in_specs, args, aliases = data_specs, data_args, {}
        tc_body = lambda *refs: body(None, *refs)
    else:
        in_specs = [pl.BlockSpec(memory_space=pl.ANY)] + data_specs
        args = (carry,) + data_args
        aliases = {0: 0}
        tc_body = body
    return pl.pallas_call(
        tc_body,
        grid=(steps,),
        in_specs=in_specs,
        out_specs=pl.BlockSpec((_BB, T, d_model),
                               lambda i, s=s, steps=steps: (s * steps + i, 0, 0)),
        out_shape=jax.ShapeDtypeStruct((B, T, d_model), jnp.float32),
        input_output_aliases=aliases,
    )(*args)


def kernel(pose_ids, motion_ids, dynamics_ids, face_ids, pose_table,
           motion_table, dynamics_table, face_table, W, b, ln_gamma, ln_beta):
    B, T = pose_ids.shape
    R = B * T
    rs = R // _SLICES
    ids = [x.reshape(-1).astype(jnp.int32)
           for x in (pose_ids, motion_ids, dynamics_ids, face_ids)]
    tables = (pose_table, motion_table, dynamics_table, face_table)
    pe = jnp.asarray(_make_pe(T, W.shape[1]))
    b2, g2, be2 = b.reshape(1, -1), ln_gamma.reshape(1, -1), ln_beta.reshape(1, -1)

    embs = []
    for s in range(_SLICES):
        ids_s = [i[s * rs:(s + 1) * rs] for i in ids]
        embs.append(_sc_gather(tables, ids_s, rs))
    out = None
    for s in range(_SLICES):
        out = _tc_post_slice(out, embs[s], W, b2, g2, be2, pe, B, T, s, _SLICES)
    return out
